# Initial kernel scaffold; baseline (speedup 1.0000x reference)
import jax
import jax.numpy as jnp
from jax.experimental import pallas as pl


def kernel(x, edge_index, W_P_w, W_P_b, W_pos, q0, k0, v0, g0, b0, q1, k1, v1, g1, b1, invW_w, invW_b):
    raise NotImplementedError("write your pallas kernel here")


# speedup vs baseline: 1.6291x; 1.6291x over previous
import functools

import jax
import jax.numpy as jnp
from jax import lax
from jax.experimental import pallas as pl
from jax.experimental.pallas import tpu as pltpu
from jax.experimental.pallas import tpu_sc as plsc

N = 10000
E = 320000
D = 128
HEAD = 4
DH = D // HEAD

NC = 2
NS = 16
NW = NC * NS
EPT = E // NW
CH = 40
NCHUNK = EPT // CH
NP = 10240
NPK = 1280
RPT = NP // NS
RFULL = RPT // CH
KPT = NPK // NS


def _f32_dot(a, b, contract):
    return lax.dot_general(a, b, (contract, ((), ())),
                           preferred_element_type=jnp.float32)


def _pre_body(x_ref, wp_ref, wpb_ref, pos_ref, q_ref, k_ref, v_ref,
              emb_ref, qn_ref, kv_ref):
    z = _f32_dot(x_ref[...], wp_ref[...], ((1,), (1,)))
    z = z + wpb_ref[...] + pos_ref[...]
    emb_ref[...] = z
    qn_ref[...] = _f32_dot(z, q_ref[...], ((1,), (0,)))
    kv_ref[:, :D] = _f32_dot(z, k_ref[...], ((1,), (0,)))
    kv_ref[:, D:] = _f32_dot(z, v_ref[...], ((1,), (0,)))


ROWB = 2000


def _tc_pre(x, wp, wpb, pos, q, k, v):
    full = lambda i: (0, 0)
    return pl.pallas_call(
        _pre_body,
        grid=(N // ROWB,),
        in_specs=[
            pl.BlockSpec((ROWB, D), lambda i: (i, 0)),
            pl.BlockSpec((D, D), full),
            pl.BlockSpec((1, D), full),
            pl.BlockSpec((1, D), full),
            pl.BlockSpec((D, D), full),
            pl.BlockSpec((D, D), full),
            pl.BlockSpec((D, D), full),
        ],
        out_specs=[
            pl.BlockSpec((ROWB, D), lambda i: (i, 0)),
            pl.BlockSpec((ROWB, D), lambda i: (i, 0)),
            pl.BlockSpec((ROWB, 2 * D), lambda i: (i, 0)),
        ],
        out_shape=[
            jax.ShapeDtypeStruct((N, D), jnp.float32),
            jax.ShapeDtypeStruct((N, D), jnp.float32),
            jax.ShapeDtypeStruct((N, 2 * D), jnp.float32),
        ],
    )(x, wp, wpb, pos, q, k, v)


def _combine_ln(wv_ref, nm_ref, emb_ref, g_ref, b_ref):
    wv = wv_ref[0] + wv_ref[1]
    nm = nm_ref[0] + nm_ref[1]
    j16 = lax.broadcasted_iota(jnp.int32, (16, D), 0)
    c128 = lax.broadcasted_iota(jnp.int32, (16, D), 1)
    expand = jnp.where(j16 == c128 // DH, 1.0, 0.0).astype(jnp.float32)
    denom = _f32_dot(nm, expand, ((1,), (0,))) + 1e-8
    out = wv / denom + emb_ref[...]
    m = jnp.mean(out, axis=1, keepdims=True)
    c = out - m
    var = jnp.mean(c * c, axis=1, keepdims=True)
    return c * lax.rsqrt(var + 1e-6) * g_ref[...] + b_ref[...]


def _mid_body(wv_ref, nm_ref, emb_ref, g_ref, b_ref, q_ref, k_ref, v_ref,
              emb2_ref, qn_ref, kv_ref):
    ln = _combine_ln(wv_ref, nm_ref, emb_ref, g_ref, b_ref)
    emb2_ref[...] = ln
    qn_ref[...] = _f32_dot(ln, q_ref[...], ((1,), (0,)))
    kv_ref[:, :D] = _f32_dot(ln, k_ref[...], ((1,), (0,)))
    kv_ref[:, D:] = _f32_dot(ln, v_ref[...], ((1,), (0,)))


def _tc_mid(wv, nm, emb, g, b, q, k, v):
    full = lambda i: (0, 0)
    full3 = lambda i: (0, i, 0)
    return pl.pallas_call(
        _mid_body,
        grid=(N // ROWB,),
        in_specs=[
            pl.BlockSpec((NC, ROWB, D), full3),
            pl.BlockSpec((NC, ROWB, 16), full3),
            pl.BlockSpec((ROWB, D), lambda i: (i, 0)),
            pl.BlockSpec((1, D), full),
            pl.BlockSpec((1, D), full),
            pl.BlockSpec((D, D), full),
            pl.BlockSpec((D, D), full),
            pl.BlockSpec((D, D), full),
        ],
        out_specs=[
            pl.BlockSpec((ROWB, D), lambda i: (i, 0)),
            pl.BlockSpec((ROWB, D), lambda i: (i, 0)),
            pl.BlockSpec((ROWB, 2 * D), lambda i: (i, 0)),
        ],
        out_shape=[
            jax.ShapeDtypeStruct((N, D), jnp.float32),
            jax.ShapeDtypeStruct((N, D), jnp.float32),
            jax.ShapeDtypeStruct((N, 2 * D), jnp.float32),
        ],
    )(wv, nm, emb, g, b, q, k, v)


def _post_body(wv_ref, nm_ref, emb_ref, g_ref, b_ref, iw_ref, ib_ref,
               ret_ref):
    ln = _combine_ln(wv_ref, nm_ref, emb_ref, g_ref, b_ref)
    ret_ref[...] = _f32_dot(ln, iw_ref[...], ((1,), (1,))) + ib_ref[...]


def _tc_post(wv, nm, emb, g, b, iw, ib):
    full = lambda i: (0, 0)
    full3 = lambda i: (0, i, 0)
    return pl.pallas_call(
        _post_body,
        grid=(N // ROWB,),
        in_specs=[
            pl.BlockSpec((NC, ROWB, D), full3),
            pl.BlockSpec((NC, ROWB, 16), full3),
            pl.BlockSpec((ROWB, D), lambda i: (i, 0)),
            pl.BlockSpec((1, D), full),
            pl.BlockSpec((1, D), full),
            pl.BlockSpec((D, D), full),
            pl.BlockSpec((1, D), full),
        ],
        out_specs=pl.BlockSpec((ROWB, D), lambda i: (i, 0)),
        out_shape=jax.ShapeDtypeStruct((N, D), jnp.float32),
    )(wv, nm, emb, g, b, iw, ib)


def _idx_body(r_ref, r8_ref, rb_ref):
    r = r_ref[...]
    r8_ref[...] = lax.shift_right_logical(r, 3)
    rb_ref[...] = lax.shift_left(jnp.bitwise_and(r, 7), 4)


def _tc_idx(rows2d):
    return pl.pallas_call(
        _idx_body,
        grid=(1,),
        in_specs=[pl.BlockSpec((E // 128, 128), lambda i: (0, 0))],
        out_specs=[
            pl.BlockSpec((E // 128, 128), lambda i: (0, 0)),
            pl.BlockSpec((E // 128, 128), lambda i: (0, 0)),
        ],
        out_shape=[
            jax.ShapeDtypeStruct((E // 128, 128), jnp.int32),
            jax.ShapeDtypeStruct((E // 128, 128), jnp.int32),
        ],
    )(rows2d)


def _sc_edges_body(qn_hbm, kv_hbm, rows_hbm, cols_hbm, r8_hbm, rb_hbm,
                   owv_hbm, onm_hbm,
                   ridx, cidx, r8idx, rbbuf, qbuf, kvbuf, wvbuf, nmbuf,
                   accwv, accnm, gsem):
    cid = lax.axis_index("c")
    sid = lax.axis_index("s")
    wid = cid * NS + sid
    row0 = sid * RPT
    _ZERO16 = jnp.zeros((16,), jnp.float32)

    def zrow(r, carry):
        for cc in range(D // 16):
            wvbuf[r, pl.ds(cc * 16, 16)] = _ZERO16
        return carry

    lax.fori_loop(0, CH, zrow, 0)

    def zcp(j, carry):
        off = row0 + j * CH
        pltpu.sync_copy(wvbuf, accwv.at[pl.ds(off, CH)])
        return carry

    lax.fori_loop(0, RFULL, zcp, 0)
    for j in range(KPT // CH):
        pltpu.sync_copy(wvbuf, accnm.at[pl.ds(sid * KPT + j * CH, CH)])
    plsc.subcore_barrier()

    def chunk(i, carry):
        base = wid * EPT + i * CH
        pltpu.sync_copy(rows_hbm.at[pl.ds(base, CH)], ridx)
        pltpu.sync_copy(cols_hbm.at[pl.ds(base, CH)], cidx)
        pltpu.sync_copy(r8_hbm.at[pl.ds(base, CH)], r8idx)
        pltpu.sync_copy(rb_hbm.at[pl.ds(base, CH)], rbbuf)
        cq = pltpu.async_copy(qn_hbm.at[ridx], qbuf, gsem)
        ckv = pltpu.async_copy(kv_hbm.at[cidx], kvbuf, gsem)
        cq.wait()
        ckv.wait()

        def edge(e, ecarry):
            att16 = _ZERO16
            for h in range(HEAD):
                p0 = (qbuf[e, pl.ds(h * DH, 16)]
                      * kvbuf[e, pl.ds(h * DH, 16)])
                p1 = (qbuf[e, pl.ds(h * DH + 16, 16)]
                      * kvbuf[e, pl.ds(h * DH + 16, 16)])
                a = jnp.sum(p0 + p1)
                a = jnp.minimum(jnp.maximum(a, -10.0), 10.0)
                ev = jnp.exp(jnp.full((16,), a, jnp.float32))
                wvbuf[e, pl.ds(h * DH, 16)] = (
                    ev * kvbuf[e, pl.ds(D + h * DH, 16)])
                wvbuf[e, pl.ds(h * DH + 16, 16)] = (
                    ev * kvbuf[e, pl.ds(D + h * DH + 16, 16)])
                att16 = att16 + jnp.where(lax.iota(jnp.int32, 16) == h,
                                          ev, 0.0)
            rbv = plsc.load_gather(rbbuf, [jnp.full((16,), e, jnp.int32)])
            for b in range(8):
                nmbuf[e, pl.ds(b * 16, 16)] = jnp.where(
                    rbv == b * 16, att16, 0.0)
            return ecarry

        lax.fori_loop(0, CH, edge, 0)
        pltpu.sync_copy(wvbuf, accwv.at[ridx], add=True)
        pltpu.sync_copy(nmbuf, accnm.at[r8idx], add=True)
        return carry

    lax.fori_loop(0, NCHUNK, chunk, 0)
    plsc.subcore_barrier()

    def cp(j, carry):
        off = row0 + j * CH
        pltpu.sync_copy(accwv.at[pl.ds(off, CH)], wvbuf)
        pltpu.sync_copy(wvbuf, owv_hbm.at[cid, pl.ds(off, CH)])
        return carry

    lax.fori_loop(0, RFULL, cp, 0)
    for j in range(KPT // CH):
        off = sid * KPT + j * CH
        pltpu.sync_copy(accnm.at[pl.ds(off, CH)], nmbuf)
        pltpu.sync_copy(nmbuf, onm_hbm.at[cid, pl.ds(off, CH)])


@functools.cache
def _sc_edges():
    return functools.partial(
        pl.kernel,
        out_type=[
            jax.ShapeDtypeStruct((NC, NP, D), jnp.float32),
            jax.ShapeDtypeStruct((NC, NPK, D), jnp.float32),
        ],
        mesh=plsc.VectorSubcoreMesh(core_axis_name="c", subcore_axis_name="s",
                                    num_cores=NC, num_subcores=NS),
        scratch_types=[
            pltpu.VMEM((CH,), jnp.int32),
            pltpu.VMEM((CH,), jnp.int32),
            pltpu.VMEM((CH,), jnp.int32),
            pltpu.VMEM((CH,), jnp.int32),
            pltpu.VMEM((CH, D), jnp.float32),
            pltpu.VMEM((CH, 2 * D), jnp.float32),
            pltpu.VMEM((CH, D), jnp.float32),
            pltpu.VMEM((CH, D), jnp.float32),
            pltpu.VMEM_SHARED((NP, D), jnp.float32),
            pltpu.VMEM_SHARED((NPK, D), jnp.float32),
            pltpu.SemaphoreType.DMA,
        ],
        compiler_params=pltpu.CompilerParams(needs_layout_passes=False),
    )(_sc_edges_body)


def kernel(x, edge_index, W_P_w, W_P_b, W_pos, q0, k0, v0, g0, b0,
           q1, k1, v1, g1, b1, invW_w, invW_b):
    rows = edge_index[0]
    cols = edge_index[1]
    wpb = W_P_b.reshape(1, D)
    g0r, b0r = g0.reshape(1, D), b0.reshape(1, D)
    g1r, b1r = g1.reshape(1, D), b1.reshape(1, D)
    ibr = invW_b.reshape(1, D)

    sc_edges = _sc_edges()
    r8_2d, rb_2d = _tc_idx(rows.reshape(E // 128, 128))
    r8 = r8_2d.reshape(E)
    rb = rb_2d.reshape(E)
    emb1, qn1, kv1 = _tc_pre(x, W_P_w, wpb, W_pos, q0, k0, v0)
    wv1, nm1 = sc_edges(qn1, kv1, rows, cols, r8, rb)
    nm1 = nm1.reshape(NC, NPK * 8, 16)
    emb2, qn2, kv2 = _tc_mid(wv1, nm1, emb1, g0r, b0r, q1, k1, v1)
    wv2, nm2 = sc_edges(qn2, kv2, rows, cols, r8, rb)
    nm2 = nm2.reshape(NC, NPK * 8, 16)
    return _tc_post(wv2, nm2, emb2, g1r, b1r, invW_w, ibr)


# speedup vs baseline: 2.1291x; 1.3069x over previous
import functools

import jax
import jax.numpy as jnp
from jax import lax
from jax.experimental import pallas as pl
from jax.experimental.pallas import tpu as pltpu
from jax.experimental.pallas import tpu_sc as plsc

N = 10000
E = 320000
D = 128
HEAD = 4
DH = D // HEAD

NC = 2
NS = 16
NW = NC * NS
EPT = E // NW
CH = 40
NCHUNK = EPT // CH
NPAIR = NCHUNK // 2
NP = 10240
NPK = 384
RPT = NP // NS
RFULL = RPT // CH
KPT = NPK // NS


def _f32_dot(a, b, contract):
    return lax.dot_general(a, b, (contract, ((), ())),
                           preferred_element_type=jnp.float32)


def _pre_body(x_ref, wp_ref, wpb_ref, pos_ref, q_ref, k_ref, v_ref,
              emb_ref, qn_ref, kv_ref):
    z = _f32_dot(x_ref[...], wp_ref[...], ((1,), (1,)))
    z = z + wpb_ref[...] + pos_ref[...]
    emb_ref[...] = z
    qn_ref[...] = _f32_dot(z, q_ref[...], ((1,), (0,)))
    kv_ref[:, :D] = _f32_dot(z, k_ref[...], ((1,), (0,)))
    kv_ref[:, D:] = _f32_dot(z, v_ref[...], ((1,), (0,)))


ROWB = 2000


def _tc_pre(x, wp, wpb, pos, q, k, v):
    full = lambda i: (0, 0)
    return pl.pallas_call(
        _pre_body,
        grid=(N // ROWB,),
        in_specs=[
            pl.BlockSpec((ROWB, D), lambda i: (i, 0)),
            pl.BlockSpec((D, D), full),
            pl.BlockSpec((1, D), full),
            pl.BlockSpec((1, D), full),
            pl.BlockSpec((D, D), full),
            pl.BlockSpec((D, D), full),
            pl.BlockSpec((D, D), full),
        ],
        out_specs=[
            pl.BlockSpec((ROWB, D), lambda i: (i, 0)),
            pl.BlockSpec((ROWB, D), lambda i: (i, 0)),
            pl.BlockSpec((ROWB, 2 * D), lambda i: (i, 0)),
        ],
        out_shape=[
            jax.ShapeDtypeStruct((N, D), jnp.float32),
            jax.ShapeDtypeStruct((N, D), jnp.float32),
            jax.ShapeDtypeStruct((N, 2 * D), jnp.float32),
        ],
    )(x, wp, wpb, pos, q, k, v)


def _combine_ln(wv_ref, nm_ref, emb_ref, g_ref, b_ref):
    wv = wv_ref[0] + wv_ref[1]
    nm = nm_ref[0] + nm_ref[1]
    j4 = lax.broadcasted_iota(jnp.int32, (HEAD, D), 0)
    c128 = lax.broadcasted_iota(jnp.int32, (HEAD, D), 1)
    expand = jnp.where(j4 == c128 // DH, 1.0, 0.0).astype(jnp.float32)
    denom = _f32_dot(nm, expand, ((1,), (0,))) + 1e-8
    out = wv / denom + emb_ref[...]
    m = jnp.mean(out, axis=1, keepdims=True)
    c = out - m
    var = jnp.mean(c * c, axis=1, keepdims=True)
    return c * lax.rsqrt(var + 1e-6) * g_ref[...] + b_ref[...]


def _mid_body(wv_ref, nm_ref, emb_ref, g_ref, b_ref, q_ref, k_ref, v_ref,
              emb2_ref, qn_ref, kv_ref):
    ln = _combine_ln(wv_ref, nm_ref, emb_ref, g_ref, b_ref)
    emb2_ref[...] = ln
    qn_ref[...] = _f32_dot(ln, q_ref[...], ((1,), (0,)))
    kv_ref[:, :D] = _f32_dot(ln, k_ref[...], ((1,), (0,)))
    kv_ref[:, D:] = _f32_dot(ln, v_ref[...], ((1,), (0,)))


def _tc_mid(wv, nm, emb, g, b, q, k, v):
    full = lambda i: (0, 0)
    full3 = lambda i: (0, i, 0)
    return pl.pallas_call(
        _mid_body,
        grid=(N // ROWB,),
        in_specs=[
            pl.BlockSpec((NC, ROWB, D), full3),
            pl.BlockSpec((NC, ROWB, HEAD), full3),
            pl.BlockSpec((ROWB, D), lambda i: (i, 0)),
            pl.BlockSpec((1, D), full),
            pl.BlockSpec((1, D), full),
            pl.BlockSpec((D, D), full),
            pl.BlockSpec((D, D), full),
            pl.BlockSpec((D, D), full),
        ],
        out_specs=[
            pl.BlockSpec((ROWB, D), lambda i: (i, 0)),
            pl.BlockSpec((ROWB, D), lambda i: (i, 0)),
            pl.BlockSpec((ROWB, 2 * D), lambda i: (i, 0)),
        ],
        out_shape=[
            jax.ShapeDtypeStruct((N, D), jnp.float32),
            jax.ShapeDtypeStruct((N, D), jnp.float32),
            jax.ShapeDtypeStruct((N, 2 * D), jnp.float32),
        ],
    )(wv, nm, emb, g, b, q, k, v)


def _post_body(wv_ref, nm_ref, emb_ref, g_ref, b_ref, iw_ref, ib_ref,
               ret_ref):
    ln = _combine_ln(wv_ref, nm_ref, emb_ref, g_ref, b_ref)
    ret_ref[...] = _f32_dot(ln, iw_ref[...], ((1,), (1,))) + ib_ref[...]


def _tc_post(wv, nm, emb, g, b, iw, ib):
    full = lambda i: (0, 0)
    full3 = lambda i: (0, i, 0)
    return pl.pallas_call(
        _post_body,
        grid=(N // ROWB,),
        in_specs=[
            pl.BlockSpec((NC, ROWB, D), full3),
            pl.BlockSpec((NC, ROWB, HEAD), full3),
            pl.BlockSpec((ROWB, D), lambda i: (i, 0)),
            pl.BlockSpec((1, D), full),
            pl.BlockSpec((1, D), full),
            pl.BlockSpec((D, D), full),
            pl.BlockSpec((1, D), full),
        ],
        out_specs=pl.BlockSpec((ROWB, D), lambda i: (i, 0)),
        out_shape=jax.ShapeDtypeStruct((N, D), jnp.float32),
    )(wv, nm, emb, g, b, iw, ib)


def _idx_body(r_ref, r8_ref, rb_ref):
    r = r_ref[...]
    r8_ref[...] = lax.shift_right_logical(r, 5)
    rb_ref[...] = lax.shift_left(jnp.bitwise_and(r, 31), 2)


def _tc_idx(rows2d):
    return pl.pallas_call(
        _idx_body,
        grid=(1,),
        in_specs=[pl.BlockSpec((E // 128, 128), lambda i: (0, 0))],
        out_specs=[
            pl.BlockSpec((E // 128, 128), lambda i: (0, 0)),
            pl.BlockSpec((E // 128, 128), lambda i: (0, 0)),
        ],
        out_shape=[
            jax.ShapeDtypeStruct((E // 128, 128), jnp.int32),
            jax.ShapeDtypeStruct((E // 128, 128), jnp.int32),
        ],
    )(rows2d)


def _issue_idx(rows_hbm, cols_hbm, r32_hbm, rb4_hbm, ridx, cidx, r32idx,
               rb4buf, s, base, isem):
    copies = [
        pltpu.async_copy(rows_hbm.at[pl.ds(base, CH)], ridx.at[s], isem),
        pltpu.async_copy(cols_hbm.at[pl.ds(base, CH)], cidx.at[s], isem),
        pltpu.async_copy(r32_hbm.at[pl.ds(base, CH)], r32idx.at[s], isem),
        pltpu.async_copy(rb4_hbm.at[pl.ds(base, CH)], rb4buf.at[s], isem),
    ]
    for c in copies:
        c.wait()


def _sc_edges_body(qn_hbm, kv_hbm, rows_hbm, cols_hbm, r32_hbm, rb4_hbm,
                   owv_hbm, onm_hbm,
                   ridx, cidx, r32idx, rb4buf, qbuf, kvbuf, wvbuf, nmbuf,
                   accwv, accnm, isem, gsem0, gsem1):
    cid = lax.axis_index("c")
    sid = lax.axis_index("s")
    wid = cid * NS + sid
    row0 = sid * RPT
    _ZERO16 = jnp.zeros((16,), jnp.float32)
    gsems = (gsem0, gsem1)

    def gathers(s):
        return (pltpu.make_async_copy(qn_hbm.at[ridx.at[s]], qbuf.at[s],
                                      gsems[s]),
                pltpu.make_async_copy(kv_hbm.at[cidx.at[s]], kvbuf.at[s],
                                      gsems[s]))

    def issue_gathers(s):
        cq, ckv = gathers(s)
        cq.start()
        ckv.start()

    def wait_gathers(s):
        cq, ckv = gathers(s)
        cq.wait()
        ckv.wait()

    def zrow(r, carry):
        for cc in range(D // 16):
            wvbuf[r, pl.ds(cc * 16, 16)] = _ZERO16
        return carry

    lax.fori_loop(0, CH, zrow, 0)

    def zcp(j, carry):
        off = row0 + j * CH
        pltpu.sync_copy(wvbuf, accwv.at[pl.ds(off, CH)])
        return carry

    lax.fori_loop(0, RFULL, zcp, 0)
    pltpu.sync_copy(wvbuf.at[pl.ds(0, KPT)], accnm.at[pl.ds(sid * KPT, KPT)])
    plsc.subcore_barrier()

    lane = lax.iota(jnp.int32, 16)
    lane4 = jnp.bitwise_and(lane, 3)
    laneb = lane - lane4

    def compute_chunk(s, i):
        def edge(e, ecarry):
            att4 = _ZERO16
            for h in range(HEAD):
                p0 = (qbuf[s, e, pl.ds(h * DH, 16)]
                      * kvbuf[s, e, pl.ds(h * DH, 16)])
                p1 = (qbuf[s, e, pl.ds(h * DH + 16, 16)]
                      * kvbuf[s, e, pl.ds(h * DH + 16, 16)])
                a = jnp.sum(p0 + p1)
                a = jnp.minimum(jnp.maximum(a, -10.0), 10.0)
                ev = jnp.exp(jnp.full((16,), a, jnp.float32))
                wvbuf[e, pl.ds(h * DH, 16)] = (
                    ev * kvbuf[s, e, pl.ds(D + h * DH, 16)])
                wvbuf[e, pl.ds(h * DH + 16, 16)] = (
                    ev * kvbuf[s, e, pl.ds(D + h * DH + 16, 16)])
                att4 = att4 + jnp.where(lane4 == h, ev, 0.0)
            rbv = plsc.load_gather(
                rb4buf, [jnp.full((16,), s, jnp.int32),
                         jnp.full((16,), e, jnp.int32)])
            for b in range(8):
                nmbuf[e, pl.ds(b * 16, 16)] = jnp.where(
                    laneb + b * 16 == rbv, att4, 0.0)
            return ecarry

        lax.fori_loop(0, CH, edge, 0)
        pltpu.sync_copy(wvbuf, accwv.at[ridx.at[s]], add=True)
        pltpu.sync_copy(nmbuf, accnm.at[r32idx.at[s]], add=True)

    _issue_idx(rows_hbm, cols_hbm, r32_hbm, rb4_hbm, ridx, cidx, r32idx,
               rb4buf, 0, wid * EPT, isem)
    issue_gathers(0)

    def pair(p, carry):
        for s in range(2):
            i = p * 2 + s
            nxt = i + 1

            def prefetch():
                _issue_idx(rows_hbm, cols_hbm, r32_hbm, rb4_hbm,
                           ridx, cidx, r32idx, rb4buf, 1 - s,
                           wid * EPT + nxt * CH, isem)
                issue_gathers(1 - s)

            if s == 0:
                prefetch()
            else:
                @pl.when(p < NPAIR - 1)
                def _():
                    prefetch()
            wait_gathers(s)
            compute_chunk(s, i)
        return carry

    lax.fori_loop(0, NPAIR, pair, 0)
    plsc.subcore_barrier()

    def cp(j, carry):
        off = row0 + j * CH
        pltpu.sync_copy(accwv.at[pl.ds(off, CH)], wvbuf)
        pltpu.sync_copy(wvbuf, owv_hbm.at[cid, pl.ds(off, CH)])
        return carry

    lax.fori_loop(0, RFULL, cp, 0)
    off = sid * KPT
    pltpu.sync_copy(accnm.at[pl.ds(off, KPT)], nmbuf.at[pl.ds(0, KPT)])
    pltpu.sync_copy(nmbuf.at[pl.ds(0, KPT)], onm_hbm.at[cid, pl.ds(off, KPT)])


@functools.cache
def _sc_edges():
    return functools.partial(
        pl.kernel,
        out_type=[
            jax.ShapeDtypeStruct((NC, NP, D), jnp.float32),
            jax.ShapeDtypeStruct((NC, NPK, D), jnp.float32),
        ],
        mesh=plsc.VectorSubcoreMesh(core_axis_name="c", subcore_axis_name="s",
                                    num_cores=NC, num_subcores=NS),
        scratch_types=[
            pltpu.VMEM((2, CH), jnp.int32),
            pltpu.VMEM((2, CH), jnp.int32),
            pltpu.VMEM((2, CH), jnp.int32),
            pltpu.VMEM((2, CH), jnp.int32),
            pltpu.VMEM((2, CH, D), jnp.float32),
            pltpu.VMEM((2, CH, 2 * D), jnp.float32),
            pltpu.VMEM((CH, D), jnp.float32),
            pltpu.VMEM((CH, D), jnp.float32),
            pltpu.VMEM_SHARED((NP, D), jnp.float32),
            pltpu.VMEM_SHARED((NPK, D), jnp.float32),
            pltpu.SemaphoreType.DMA,
            pltpu.SemaphoreType.DMA,
            pltpu.SemaphoreType.DMA,
        ],
        compiler_params=pltpu.CompilerParams(needs_layout_passes=False),
    )(_sc_edges_body)


def kernel(x, edge_index, W_P_w, W_P_b, W_pos, q0, k0, v0, g0, b0,
           q1, k1, v1, g1, b1, invW_w, invW_b):
    rows = edge_index[0]
    cols = edge_index[1]
    wpb = W_P_b.reshape(1, D)
    g0r, b0r = g0.reshape(1, D), b0.reshape(1, D)
    g1r, b1r = g1.reshape(1, D), b1.reshape(1, D)
    ibr = invW_b.reshape(1, D)

    sc_edges = _sc_edges()
    r32_2d, rb4_2d = _tc_idx(rows.reshape(E // 128, 128))
    r32 = r32_2d.reshape(E)
    rb4 = rb4_2d.reshape(E)
    emb1, qn1, kv1 = _tc_pre(x, W_P_w, wpb, W_pos, q0, k0, v0)
    wv1, nm1 = sc_edges(qn1, kv1, rows, cols, r32, rb4)
    nm1 = nm1.reshape(NC, NPK * 32, HEAD)
    emb2, qn2, kv2 = _tc_mid(wv1, nm1, emb1, g0r, b0r, q1, k1, v1)
    wv2, nm2 = sc_edges(qn2, kv2, rows, cols, r32, rb4)
    nm2 = nm2.reshape(NC, NPK * 32, HEAD)
    return _tc_post(wv2, nm2, emb2, g1r, b1r, invW_w, ibr)


# speedup vs baseline: 3.8587x; 1.8124x over previous
import functools

import jax
import jax.numpy as jnp
from jax import lax
from jax.experimental import pallas as pl
from jax.experimental.pallas import tpu as pltpu
from jax.experimental.pallas import tpu_sc as plsc

N = 10000
E = 320000
D = 128
HEAD = 4
DH = D // HEAD

NC = 2
NS = 16
NW = NC * NS
EPT = E // NW
CH = 40
NCHUNK = EPT // CH
NPAIR = NCHUNK // 2
NP = 10240
NPK = 384
RPT = NP // NS
RFULL = RPT // CH
KPT = NPK // NS


def _f32_dot(a, b, contract):
    return lax.dot_general(a, b, (contract, ((), ())),
                           preferred_element_type=jnp.float32)


def _pre_body(x_ref, wp_ref, wpb_ref, pos_ref, q_ref, k_ref, v_ref,
              emb_ref, qn_ref, kv_ref):
    z = _f32_dot(x_ref[...], wp_ref[...], ((1,), (1,)))
    z = z + wpb_ref[...] + pos_ref[...]
    emb_ref[...] = z
    qn_ref[...] = _f32_dot(z, q_ref[...], ((1,), (0,)))
    kv_ref[:, :D] = _f32_dot(z, k_ref[...], ((1,), (0,)))
    kv_ref[:, D:] = _f32_dot(z, v_ref[...], ((1,), (0,)))


ROWB = 2000


def _tc_pre(x, wp, wpb, pos, q, k, v):
    full = lambda i: (0, 0)
    return pl.pallas_call(
        _pre_body,
        grid=(N // ROWB,),
        in_specs=[
            pl.BlockSpec((ROWB, D), lambda i: (i, 0)),
            pl.BlockSpec((D, D), full),
            pl.BlockSpec((1, D), full),
            pl.BlockSpec((1, D), full),
            pl.BlockSpec((D, D), full),
            pl.BlockSpec((D, D), full),
            pl.BlockSpec((D, D), full),
        ],
        out_specs=[
            pl.BlockSpec((ROWB, D), lambda i: (i, 0)),
            pl.BlockSpec((ROWB, D), lambda i: (i, 0)),
            pl.BlockSpec((ROWB, 2 * D), lambda i: (i, 0)),
        ],
        out_shape=[
            jax.ShapeDtypeStruct((N, D), jnp.float32),
            jax.ShapeDtypeStruct((N, D), jnp.float32),
            jax.ShapeDtypeStruct((N, 2 * D), jnp.float32),
        ],
    )(x, wp, wpb, pos, q, k, v)


def _combine_ln(wv_ref, nm_ref, emb_ref, g_ref, b_ref):
    wv = wv_ref[0] + wv_ref[1]
    nm = nm_ref[0] + nm_ref[1]
    j4 = lax.broadcasted_iota(jnp.int32, (HEAD, D), 0)
    c128 = lax.broadcasted_iota(jnp.int32, (HEAD, D), 1)
    expand = jnp.where(j4 == c128 // DH, 1.0, 0.0).astype(jnp.float32)
    denom = _f32_dot(nm, expand, ((1,), (0,))) + 1e-8
    out = wv / denom + emb_ref[...]
    m = jnp.mean(out, axis=1, keepdims=True)
    c = out - m
    var = jnp.mean(c * c, axis=1, keepdims=True)
    return c * lax.rsqrt(var + 1e-6) * g_ref[...] + b_ref[...]


def _mid_body(wv_ref, nm_ref, emb_ref, g_ref, b_ref, q_ref, k_ref, v_ref,
              emb2_ref, qn_ref, kv_ref):
    ln = _combine_ln(wv_ref, nm_ref, emb_ref, g_ref, b_ref)
    emb2_ref[...] = ln
    qn_ref[...] = _f32_dot(ln, q_ref[...], ((1,), (0,)))
    kv_ref[:, :D] = _f32_dot(ln, k_ref[...], ((1,), (0,)))
    kv_ref[:, D:] = _f32_dot(ln, v_ref[...], ((1,), (0,)))


def _tc_mid(wv, nm, emb, g, b, q, k, v):
    full = lambda i: (0, 0)
    full3 = lambda i: (0, i, 0)
    return pl.pallas_call(
        _mid_body,
        grid=(N // ROWB,),
        in_specs=[
            pl.BlockSpec((NC, ROWB, D), full3),
            pl.BlockSpec((NC, ROWB, HEAD), full3),
            pl.BlockSpec((ROWB, D), lambda i: (i, 0)),
            pl.BlockSpec((1, D), full),
            pl.BlockSpec((1, D), full),
            pl.BlockSpec((D, D), full),
            pl.BlockSpec((D, D), full),
            pl.BlockSpec((D, D), full),
        ],
        out_specs=[
            pl.BlockSpec((ROWB, D), lambda i: (i, 0)),
            pl.BlockSpec((ROWB, D), lambda i: (i, 0)),
            pl.BlockSpec((ROWB, 2 * D), lambda i: (i, 0)),
        ],
        out_shape=[
            jax.ShapeDtypeStruct((N, D), jnp.float32),
            jax.ShapeDtypeStruct((N, D), jnp.float32),
            jax.ShapeDtypeStruct((N, 2 * D), jnp.float32),
        ],
    )(wv, nm, emb, g, b, q, k, v)


def _post_body(wv_ref, nm_ref, emb_ref, g_ref, b_ref, iw_ref, ib_ref,
               ret_ref):
    ln = _combine_ln(wv_ref, nm_ref, emb_ref, g_ref, b_ref)
    ret_ref[...] = _f32_dot(ln, iw_ref[...], ((1,), (1,))) + ib_ref[...]


def _tc_post(wv, nm, emb, g, b, iw, ib):
    full = lambda i: (0, 0)
    full3 = lambda i: (0, i, 0)
    return pl.pallas_call(
        _post_body,
        grid=(N // ROWB,),
        in_specs=[
            pl.BlockSpec((NC, ROWB, D), full3),
            pl.BlockSpec((NC, ROWB, HEAD), full3),
            pl.BlockSpec((ROWB, D), lambda i: (i, 0)),
            pl.BlockSpec((1, D), full),
            pl.BlockSpec((1, D), full),
            pl.BlockSpec((D, D), full),
            pl.BlockSpec((1, D), full),
        ],
        out_specs=pl.BlockSpec((ROWB, D), lambda i: (i, 0)),
        out_shape=jax.ShapeDtypeStruct((N, D), jnp.float32),
    )(wv, nm, emb, g, b, iw, ib)


def _idx_body(r_ref, r8_ref, rb_ref):
    r = r_ref[...]
    r8_ref[...] = lax.shift_right_logical(r, 5)
    rb_ref[...] = lax.shift_left(jnp.bitwise_and(r, 31), 2)


def _tc_idx(rows2d):
    return pl.pallas_call(
        _idx_body,
        grid=(1,),
        in_specs=[pl.BlockSpec((E // 128, 128), lambda i: (0, 0))],
        out_specs=[
            pl.BlockSpec((E // 128, 128), lambda i: (0, 0)),
            pl.BlockSpec((E // 128, 128), lambda i: (0, 0)),
        ],
        out_shape=[
            jax.ShapeDtypeStruct((E // 128, 128), jnp.int32),
            jax.ShapeDtypeStruct((E // 128, 128), jnp.int32),
        ],
    )(rows2d)


def _issue_idx(rows_hbm, cols_hbm, r32_hbm, rb4_hbm, ridx, cidx, r32idx,
               rb4buf, s, base, isem):
    copies = [
        pltpu.async_copy(rows_hbm.at[pl.ds(base, CH)], ridx.at[s], isem),
        pltpu.async_copy(cols_hbm.at[pl.ds(base, CH)], cidx.at[s], isem),
        pltpu.async_copy(r32_hbm.at[pl.ds(base, CH)], r32idx.at[s], isem),
        pltpu.async_copy(rb4_hbm.at[pl.ds(base, CH)], rb4buf.at[s], isem),
    ]
    for c in copies:
        c.wait()


def _sc_edges_body(qn_hbm, kv_hbm, rows_hbm, cols_hbm, r32_hbm, rb4_hbm,
                   owv_hbm, onm_hbm,
                   ridx, cidx, r32idx, rb4buf, qbuf, kvbuf, wvbuf, nmbuf,
                   accwv, accnm, isem, gsem0, gsem1):
    cid = lax.axis_index("c")
    sid = lax.axis_index("s")
    wid = cid * NS + sid
    row0 = sid * RPT
    _ZERO16 = jnp.zeros((16,), jnp.float32)
    gsems = (gsem0, gsem1)

    def gathers(s):
        return (pltpu.make_async_copy(qn_hbm.at[ridx.at[s]], qbuf.at[s],
                                      gsems[s]),
                pltpu.make_async_copy(kv_hbm.at[cidx.at[s]], kvbuf.at[s],
                                      gsems[s]))

    def issue_gathers(s):
        cq, ckv = gathers(s)
        cq.start()
        ckv.start()

    def wait_gathers(s):
        cq, ckv = gathers(s)
        cq.wait()
        ckv.wait()

    def zrow(r, carry):
        for cc in range(D // 16):
            wvbuf[r, pl.ds(cc * 16, 16)] = _ZERO16
        return carry

    lax.fori_loop(0, CH, zrow, 0)

    def zcp(j, carry):
        off = row0 + j * CH
        pltpu.sync_copy(wvbuf, accwv.at[pl.ds(off, CH)])
        return carry

    lax.fori_loop(0, RFULL, zcp, 0)
    pltpu.sync_copy(wvbuf.at[pl.ds(0, KPT)], accnm.at[pl.ds(sid * KPT, KPT)])
    plsc.subcore_barrier()

    lane = lax.iota(jnp.int32, 16)
    lane4 = jnp.bitwise_and(lane, 3)
    laneb = lane - lane4

    def compute_chunk(s, i):
        def do_edge(e):
            sums = []
            for h in range(HEAD):
                p0 = (qbuf[s, e, pl.ds(h * DH, 16)]
                      * kvbuf[s, e, pl.ds(h * DH, 16)])
                p1 = (qbuf[s, e, pl.ds(h * DH + 16, 16)]
                      * kvbuf[s, e, pl.ds(h * DH + 16, 16)])
                sums.append(jnp.sum(p0 + p1))
            evs = []
            for h in range(HEAD):
                a = jnp.minimum(jnp.maximum(sums[h], -10.0), 10.0)
                evs.append(jnp.exp(jnp.full((16,), a, jnp.float32)))
            att4 = _ZERO16
            for h in range(HEAD):
                ev = evs[h]
                wvbuf[e, pl.ds(h * DH, 16)] = (
                    ev * kvbuf[s, e, pl.ds(D + h * DH, 16)])
                wvbuf[e, pl.ds(h * DH + 16, 16)] = (
                    ev * kvbuf[s, e, pl.ds(D + h * DH + 16, 16)])
                att4 = att4 + jnp.where(lane4 == h, ev, 0.0)
            rbv = plsc.load_gather(
                rb4buf, [jnp.full((16,), s, jnp.int32),
                         jnp.full((16,), e, jnp.int32)])
            for b in range(8):
                nmbuf[e, pl.ds(b * 16, 16)] = jnp.where(
                    laneb + b * 16 == rbv, att4, 0.0)

        def edge2(e2, ecarry):
            do_edge(e2 * 2)
            do_edge(e2 * 2 + 1)
            return ecarry

        lax.fori_loop(0, CH // 2, edge2, 0)
        pltpu.sync_copy(wvbuf, accwv.at[ridx.at[s]], add=True)
        pltpu.sync_copy(nmbuf, accnm.at[r32idx.at[s]], add=True)

    _issue_idx(rows_hbm, cols_hbm, r32_hbm, rb4_hbm, ridx, cidx, r32idx,
               rb4buf, 0, wid * EPT, isem)
    issue_gathers(0)

    def pair(p, carry):
        for s in range(2):
            i = p * 2 + s
            nxt = i + 1

            def prefetch():
                _issue_idx(rows_hbm, cols_hbm, r32_hbm, rb4_hbm,
                           ridx, cidx, r32idx, rb4buf, 1 - s,
                           wid * EPT + nxt * CH, isem)
                issue_gathers(1 - s)

            if s == 0:
                prefetch()
            else:
                @pl.when(p < NPAIR - 1)
                def _():
                    prefetch()
            wait_gathers(s)
            compute_chunk(s, i)
        return carry

    lax.fori_loop(0, NPAIR, pair, 0)
    plsc.subcore_barrier()

    def cp(j, carry):
        off = row0 + j * CH
        pltpu.sync_copy(accwv.at[pl.ds(off, CH)], wvbuf)
        pltpu.sync_copy(wvbuf, owv_hbm.at[cid, pl.ds(off, CH)])
        return carry

    lax.fori_loop(0, RFULL, cp, 0)
    off = sid * KPT
    pltpu.sync_copy(accnm.at[pl.ds(off, KPT)], nmbuf.at[pl.ds(0, KPT)])
    pltpu.sync_copy(nmbuf.at[pl.ds(0, KPT)], onm_hbm.at[cid, pl.ds(off, KPT)])


@functools.cache
def _sc_edges():
    return functools.partial(
        pl.kernel,
        out_type=[
            jax.ShapeDtypeStruct((NC, NP, D), jnp.float32),
            jax.ShapeDtypeStruct((NC, NPK, D), jnp.float32),
        ],
        mesh=plsc.VectorSubcoreMesh(core_axis_name="c", subcore_axis_name="s",
                                    num_cores=NC, num_subcores=NS),
        scratch_types=[
            pltpu.VMEM((2, CH), jnp.int32),
            pltpu.VMEM((2, CH), jnp.int32),
            pltpu.VMEM((2, CH), jnp.int32),
            pltpu.VMEM((2, CH), jnp.int32),
            pltpu.VMEM((2, CH, D), jnp.float32),
            pltpu.VMEM((2, CH, 2 * D), jnp.float32),
            pltpu.VMEM((CH, D), jnp.float32),
            pltpu.VMEM((CH, D), jnp.float32),
            pltpu.VMEM_SHARED((NP, D), jnp.float32),
            pltpu.VMEM_SHARED((NPK, D), jnp.float32),
            pltpu.SemaphoreType.DMA,
            pltpu.SemaphoreType.DMA,
            pltpu.SemaphoreType.DMA,
        ],
        compiler_params=pltpu.CompilerParams(needs_layout_passes=False),
    )(_sc_edges_body)


def kernel(x, edge_index, W_P_w, W_P_b, W_pos, q0, k0, v0, g0, b0,
           q1, k1, v1, g1, b1, invW_w, invW_b):
    rows = edge_index[0]
    cols = edge_index[1]
    wpb = W_P_b.reshape(1, D)
    g0r, b0r = g0.reshape(1, D), b0.reshape(1, D)
    g1r, b1r = g1.reshape(1, D), b1.reshape(1, D)
    ibr = invW_b.reshape(1, D)

    sc_edges = _sc_edges()
    r32_2d, rb4_2d = _tc_idx(rows.reshape(E // 128, 128))
    r32 = r32_2d.reshape(E)
    rb4 = rb4_2d.reshape(E)
    emb1, qn1, kv1 = _tc_pre(x, W_P_w, wpb, W_pos, q0, k0, v0)
    wv1, nm1 = sc_edges(qn1, kv1, rows, cols, r32, rb4)
    nm1 = nm1.reshape(NC, NPK * 32, HEAD)
    emb2, qn2, kv2 = _tc_mid(wv1, nm1, emb1, g0r, b0r, q1, k1, v1)
    wv2, nm2 = sc_edges(qn2, kv2, rows, cols, r32, rb4)
    nm2 = nm2.reshape(NC, NPK * 32, HEAD)
    return _tc_post(wv2, nm2, emb2, g1r, b1r, invW_w, ibr)


# speedup vs baseline: 3.9402x; 1.0211x over previous
import functools

import jax
import jax.numpy as jnp
from jax import lax
from jax.experimental import pallas as pl
from jax.experimental.pallas import tpu as pltpu
from jax.experimental.pallas import tpu_sc as plsc

N = 10000
E = 320000
D = 128
HEAD = 4
DH = D // HEAD

NC = 2
NS = 16
NW = NC * NS
EPT = E // NW
CH = 40
NCHUNK = EPT // CH
NPAIR = NCHUNK // 2
NP = 10240
NPK = 384
RPT = NP // NS
RFULL = RPT // CH
KPT = NPK // NS


def _f32_dot(a, b, contract):
    return lax.dot_general(a, b, (contract, ((), ())),
                           preferred_element_type=jnp.float32)


def _pre_body(x_ref, wp_ref, wpb_ref, pos_ref, q_ref, k_ref, v_ref,
              emb_ref, qn_ref, kv_ref):
    z = _f32_dot(x_ref[...], wp_ref[...], ((1,), (1,)))
    z = z + wpb_ref[...] + pos_ref[...]
    emb_ref[...] = z
    qn_ref[...] = _f32_dot(z, q_ref[...], ((1,), (0,)))
    kv_ref[:, :D] = _f32_dot(z, k_ref[...], ((1,), (0,)))
    kv_ref[:, D:] = _f32_dot(z, v_ref[...], ((1,), (0,)))


ROWB = 2000


def _tc_pre(x, wp, wpb, pos, q, k, v):
    full = lambda i: (0, 0)
    return pl.pallas_call(
        _pre_body,
        grid=(N // ROWB,),
        in_specs=[
            pl.BlockSpec((ROWB, D), lambda i: (i, 0)),
            pl.BlockSpec((D, D), full),
            pl.BlockSpec((1, D), full),
            pl.BlockSpec((1, D), full),
            pl.BlockSpec((D, D), full),
            pl.BlockSpec((D, D), full),
            pl.BlockSpec((D, D), full),
        ],
        out_specs=[
            pl.BlockSpec((ROWB, D), lambda i: (i, 0)),
            pl.BlockSpec((ROWB, D), lambda i: (i, 0)),
            pl.BlockSpec((ROWB, 2 * D), lambda i: (i, 0)),
        ],
        out_shape=[
            jax.ShapeDtypeStruct((N, D), jnp.float32),
            jax.ShapeDtypeStruct((N, D), jnp.float32),
            jax.ShapeDtypeStruct((N, 2 * D), jnp.float32),
        ],
    )(x, wp, wpb, pos, q, k, v)


def _combine_ln(wv_ref, nm_ref, emb_ref, g_ref, b_ref):
    wv = wv_ref[0] + wv_ref[1]
    nm = nm_ref[0] + nm_ref[1]
    j4 = lax.broadcasted_iota(jnp.int32, (HEAD, D), 0)
    c128 = lax.broadcasted_iota(jnp.int32, (HEAD, D), 1)
    expand = jnp.where(j4 == c128 // DH, 1.0, 0.0).astype(jnp.float32)
    denom = _f32_dot(nm, expand, ((1,), (0,))) + 1e-8
    out = wv / denom + emb_ref[...]
    m = jnp.mean(out, axis=1, keepdims=True)
    c = out - m
    var = jnp.mean(c * c, axis=1, keepdims=True)
    return c * lax.rsqrt(var + 1e-6) * g_ref[...] + b_ref[...]


def _mid_body(wv_ref, nm_ref, emb_ref, g_ref, b_ref, q_ref, k_ref, v_ref,
              emb2_ref, qn_ref, kv_ref):
    ln = _combine_ln(wv_ref, nm_ref, emb_ref, g_ref, b_ref)
    emb2_ref[...] = ln
    qn_ref[...] = _f32_dot(ln, q_ref[...], ((1,), (0,)))
    kv_ref[:, :D] = _f32_dot(ln, k_ref[...], ((1,), (0,)))
    kv_ref[:, D:] = _f32_dot(ln, v_ref[...], ((1,), (0,)))


def _tc_mid(wv, nm, emb, g, b, q, k, v):
    full = lambda i: (0, 0)
    full3 = lambda i: (0, i, 0)
    return pl.pallas_call(
        _mid_body,
        grid=(N // ROWB,),
        in_specs=[
            pl.BlockSpec((NC, ROWB, D), full3),
            pl.BlockSpec((NC, ROWB, HEAD), full3),
            pl.BlockSpec((ROWB, D), lambda i: (i, 0)),
            pl.BlockSpec((1, D), full),
            pl.BlockSpec((1, D), full),
            pl.BlockSpec((D, D), full),
            pl.BlockSpec((D, D), full),
            pl.BlockSpec((D, D), full),
        ],
        out_specs=[
            pl.BlockSpec((ROWB, D), lambda i: (i, 0)),
            pl.BlockSpec((ROWB, D), lambda i: (i, 0)),
            pl.BlockSpec((ROWB, 2 * D), lambda i: (i, 0)),
        ],
        out_shape=[
            jax.ShapeDtypeStruct((N, D), jnp.float32),
            jax.ShapeDtypeStruct((N, D), jnp.float32),
            jax.ShapeDtypeStruct((N, 2 * D), jnp.float32),
        ],
    )(wv, nm, emb, g, b, q, k, v)


def _post_body(wv_ref, nm_ref, emb_ref, g_ref, b_ref, iw_ref, ib_ref,
               ret_ref):
    ln = _combine_ln(wv_ref, nm_ref, emb_ref, g_ref, b_ref)
    ret_ref[...] = _f32_dot(ln, iw_ref[...], ((1,), (1,))) + ib_ref[...]


def _tc_post(wv, nm, emb, g, b, iw, ib):
    full = lambda i: (0, 0)
    full3 = lambda i: (0, i, 0)
    return pl.pallas_call(
        _post_body,
        grid=(N // ROWB,),
        in_specs=[
            pl.BlockSpec((NC, ROWB, D), full3),
            pl.BlockSpec((NC, ROWB, HEAD), full3),
            pl.BlockSpec((ROWB, D), lambda i: (i, 0)),
            pl.BlockSpec((1, D), full),
            pl.BlockSpec((1, D), full),
            pl.BlockSpec((D, D), full),
            pl.BlockSpec((1, D), full),
        ],
        out_specs=pl.BlockSpec((ROWB, D), lambda i: (i, 0)),
        out_shape=jax.ShapeDtypeStruct((N, D), jnp.float32),
    )(wv, nm, emb, g, b, iw, ib)


def _idx_body(r_ref, r8_ref, rb_ref):
    r = r_ref[...]
    r8_ref[...] = lax.shift_right_logical(r, 5)
    rb_ref[...] = lax.shift_left(jnp.bitwise_and(r, 31), 2)


def _tc_idx(rows2d):
    return pl.pallas_call(
        _idx_body,
        grid=(1,),
        in_specs=[pl.BlockSpec((E // 128, 128), lambda i: (0, 0))],
        out_specs=[
            pl.BlockSpec((E // 128, 128), lambda i: (0, 0)),
            pl.BlockSpec((E // 128, 128), lambda i: (0, 0)),
        ],
        out_shape=[
            jax.ShapeDtypeStruct((E // 128, 128), jnp.int32),
            jax.ShapeDtypeStruct((E // 128, 128), jnp.int32),
        ],
    )(rows2d)


def _issue_idx(rows_hbm, cols_hbm, r32_hbm, rb4_hbm, ridx, cidx, r32idx,
               rb4buf, s, base, isem):
    copies = [
        pltpu.async_copy(rows_hbm.at[pl.ds(base, CH)], ridx.at[s], isem),
        pltpu.async_copy(cols_hbm.at[pl.ds(base, CH)], cidx.at[s], isem),
        pltpu.async_copy(r32_hbm.at[pl.ds(base, CH)], r32idx.at[s], isem),
        pltpu.async_copy(rb4_hbm.at[pl.ds(base, CH)], rb4buf.at[s], isem),
    ]
    for c in copies:
        c.wait()


def _sc_edges_body(qn_hbm, kv_hbm, rows_hbm, cols_hbm, r32_hbm, rb4_hbm,
                   owv_hbm, onm_hbm,
                   ridx, cidx, r32idx, rb4buf, qbuf, kvbuf, wvbuf, nmbuf,
                   accwv, accnm, isem, gsem0, gsem1):
    cid = lax.axis_index("c")
    sid = lax.axis_index("s")
    wid = cid * NS + sid
    row0 = sid * RPT
    _ZERO16 = jnp.zeros((16,), jnp.float32)
    gsems = (gsem0, gsem1)

    def gathers(s):
        return (pltpu.make_async_copy(qn_hbm.at[ridx.at[s]], qbuf.at[s],
                                      gsems[s]),
                pltpu.make_async_copy(kv_hbm.at[cidx.at[s]], kvbuf.at[s],
                                      gsems[s]))

    def issue_gathers(s):
        cq, ckv = gathers(s)
        cq.start()
        ckv.start()

    def wait_gathers(s):
        cq, ckv = gathers(s)
        cq.wait()
        ckv.wait()

    def zrow(r, carry):
        for cc in range(D // 16):
            wvbuf[r, pl.ds(cc * 16, 16)] = _ZERO16
        return carry

    lax.fori_loop(0, CH, zrow, 0)

    def zcp(j, carry):
        off = row0 + j * CH
        pltpu.sync_copy(wvbuf, accwv.at[pl.ds(off, CH)])
        return carry

    lax.fori_loop(0, RFULL, zcp, 0)
    pltpu.sync_copy(wvbuf.at[pl.ds(0, KPT)], accnm.at[pl.ds(sid * KPT, KPT)])
    plsc.subcore_barrier()

    lane = lax.iota(jnp.int32, 16)
    lane4 = jnp.bitwise_and(lane, 3)
    laneb = lane - lane4

    def compute_chunk(s, i):
        def do_edge(e):
            sums = []
            for h in range(HEAD):
                p0 = (qbuf[s, e, pl.ds(h * DH, 16)]
                      * kvbuf[s, e, pl.ds(h * DH, 16)])
                p1 = (qbuf[s, e, pl.ds(h * DH + 16, 16)]
                      * kvbuf[s, e, pl.ds(h * DH + 16, 16)])
                sums.append(jnp.sum(p0 + p1))
            evs = []
            for h in range(HEAD):
                a = jnp.minimum(jnp.maximum(sums[h], -10.0), 10.0)
                evs.append(jnp.exp(jnp.full((16,), a, jnp.float32)))
            att4 = _ZERO16
            for h in range(HEAD):
                ev = evs[h]
                wvbuf[e, pl.ds(h * DH, 16)] = (
                    ev * kvbuf[s, e, pl.ds(D + h * DH, 16)])
                wvbuf[e, pl.ds(h * DH + 16, 16)] = (
                    ev * kvbuf[s, e, pl.ds(D + h * DH + 16, 16)])
                att4 = att4 + jnp.where(lane4 == h, ev, 0.0)
            rbv = plsc.load_gather(
                rb4buf, [jnp.full((16,), s, jnp.int32),
                         jnp.full((16,), e, jnp.int32)])
            for b in range(8):
                nmbuf[e, pl.ds(b * 16, 16)] = jnp.where(
                    laneb + b * 16 == rbv, att4, 0.0)

        def edge4(e4, ecarry):
            for ee in range(4):
                do_edge(e4 * 4 + ee)
            return ecarry

        lax.fori_loop(0, CH // 4, edge4, 0)
        cw = pltpu.async_copy(wvbuf, accwv.at[ridx.at[s]], isem, add=True)
        cn = pltpu.async_copy(nmbuf, accnm.at[r32idx.at[s]], isem, add=True)
        cw.wait()
        cn.wait()

    _issue_idx(rows_hbm, cols_hbm, r32_hbm, rb4_hbm, ridx, cidx, r32idx,
               rb4buf, 0, wid * EPT, isem)
    issue_gathers(0)

    def pair(p, carry):
        for s in range(2):
            i = p * 2 + s
            nxt = i + 1

            def prefetch():
                _issue_idx(rows_hbm, cols_hbm, r32_hbm, rb4_hbm,
                           ridx, cidx, r32idx, rb4buf, 1 - s,
                           wid * EPT + nxt * CH, isem)
                issue_gathers(1 - s)

            if s == 0:
                prefetch()
            else:
                @pl.when(p < NPAIR - 1)
                def _():
                    prefetch()
            wait_gathers(s)
            compute_chunk(s, i)
        return carry

    lax.fori_loop(0, NPAIR, pair, 0)
    plsc.subcore_barrier()

    def cp(j, carry):
        off = row0 + j * CH
        pltpu.sync_copy(accwv.at[pl.ds(off, CH)], wvbuf)
        pltpu.sync_copy(wvbuf, owv_hbm.at[cid, pl.ds(off, CH)])
        return carry

    lax.fori_loop(0, RFULL, cp, 0)
    off = sid * KPT
    pltpu.sync_copy(accnm.at[pl.ds(off, KPT)], nmbuf.at[pl.ds(0, KPT)])
    pltpu.sync_copy(nmbuf.at[pl.ds(0, KPT)], onm_hbm.at[cid, pl.ds(off, KPT)])


@functools.cache
def _sc_edges():
    return functools.partial(
        pl.kernel,
        out_type=[
            jax.ShapeDtypeStruct((NC, NP, D), jnp.float32),
            jax.ShapeDtypeStruct((NC, NPK, D), jnp.float32),
        ],
        mesh=plsc.VectorSubcoreMesh(core_axis_name="c", subcore_axis_name="s",
                                    num_cores=NC, num_subcores=NS),
        scratch_types=[
            pltpu.VMEM((2, CH), jnp.int32),
            pltpu.VMEM((2, CH), jnp.int32),
            pltpu.VMEM((2, CH), jnp.int32),
            pltpu.VMEM((2, CH), jnp.int32),
            pltpu.VMEM((2, CH, D), jnp.float32),
            pltpu.VMEM((2, CH, 2 * D), jnp.float32),
            pltpu.VMEM((CH, D), jnp.float32),
            pltpu.VMEM((CH, D), jnp.float32),
            pltpu.VMEM_SHARED((NP, D), jnp.float32),
            pltpu.VMEM_SHARED((NPK, D), jnp.float32),
            pltpu.SemaphoreType.DMA,
            pltpu.SemaphoreType.DMA,
            pltpu.SemaphoreType.DMA,
        ],
        compiler_params=pltpu.CompilerParams(needs_layout_passes=False),
    )(_sc_edges_body)


def kernel(x, edge_index, W_P_w, W_P_b, W_pos, q0, k0, v0, g0, b0,
           q1, k1, v1, g1, b1, invW_w, invW_b):
    rows = edge_index[0]
    cols = edge_index[1]
    wpb = W_P_b.reshape(1, D)
    g0r, b0r = g0.reshape(1, D), b0.reshape(1, D)
    g1r, b1r = g1.reshape(1, D), b1.reshape(1, D)
    ibr = invW_b.reshape(1, D)

    sc_edges = _sc_edges()
    r32_2d, rb4_2d = _tc_idx(rows.reshape(E // 128, 128))
    r32 = r32_2d.reshape(E)
    rb4 = rb4_2d.reshape(E)
    emb1, qn1, kv1 = _tc_pre(x, W_P_w, wpb, W_pos, q0, k0, v0)
    wv1, nm1 = sc_edges(qn1, kv1, rows, cols, r32, rb4)
    nm1 = nm1.reshape(NC, NPK * 32, HEAD)
    emb2, qn2, kv2 = _tc_mid(wv1, nm1, emb1, g0r, b0r, q1, k1, v1)
    wv2, nm2 = sc_edges(qn2, kv2, rows, cols, r32, rb4)
    nm2 = nm2.reshape(NC, NPK * 32, HEAD)
    return _tc_post(wv2, nm2, emb2, g1r, b1r, invW_w, ibr)
